# baseline (speedup 1.0000x reference)
import functools

import jax
import jax.numpy as jnp
from jax import lax
from jax.experimental import pallas as pl
from jax.experimental.pallas import tpu as pltpu
from jax.experimental.pallas import tpu_sc as plsc

BATCH = 16384
EMBED = 64
NC = 2
NS = 16
NW = NC * NS
B_PER_W = BATCH // NW
CHUNK = 128
NCHUNK = B_PER_W // CHUNK


def _gather_body(uidx_hbm, bidx_hbm, utab_hbm, btab_hbm, u_out, v_out,
                 idx_v, rows_v, sem):
    wid = lax.axis_index("s") * NC + lax.axis_index("c")
    base = wid * B_PER_W
    for tab_hbm, idx_hbm, out in ((utab_hbm, uidx_hbm, u_out),
                                  (btab_hbm, bidx_hbm, v_out)):
        pltpu.sync_copy(idx_hbm.at[pl.ds(base, B_PER_W)], idx_v)
        copies = []
        for j in range(NCHUNK):
            copies.append(pltpu.async_copy(
                tab_hbm.at[idx_v.at[pl.ds(j * CHUNK, CHUNK)]],
                rows_v.at[pl.ds(j * CHUNK, CHUNK)], sem))
        for c in copies:
            c.wait()
        pltpu.sync_copy(rows_v, out.at[pl.ds(base, B_PER_W)])


@functools.cache
def _make_gather():
    return functools.partial(
        pl.kernel,
        mesh=plsc.VectorSubcoreMesh(core_axis_name="c", subcore_axis_name="s"),
        out_type=[
            jax.ShapeDtypeStruct((BATCH, 2 * EMBED), jnp.float32),
            jax.ShapeDtypeStruct((BATCH, 2 * EMBED), jnp.float32),
        ],
        scratch_types=[
            pltpu.VMEM((B_PER_W,), jnp.int32),
            pltpu.VMEM((B_PER_W, 2 * EMBED), jnp.float32),
            pltpu.SemaphoreType.DMA,
        ],
    )(_gather_body)


def _bn_relu(h, g, be, eps=1e-5):
    mean = jnp.mean(h, axis=0, keepdims=True)
    c = h - mean
    var = jnp.mean(c * c, axis=0, keepdims=True)
    return jnp.maximum(c * lax.rsqrt(var + eps) * g + be, 0.0)


def _mlp_body(ru_ref, rv_ref, pu_ref, pv_ref, w1a_ref, w1b_ref, b1_ref,
              g1_ref, be1_ref, w2_ref, b2_ref, g2_ref, be2_ref, w3_ref,
              b3_ref, out_ref):
    u = jnp.where(pu_ref[...] == 1, ru_ref[:, EMBED:], ru_ref[:, :EMBED])
    v = jnp.where(pv_ref[...] == 1, rv_ref[:, EMBED:], rv_ref[:, :EMBED])
    h = (jnp.dot(u, w1a_ref[...], preferred_element_type=jnp.float32)
         + jnp.dot(v, w1b_ref[...], preferred_element_type=jnp.float32)
         + b1_ref[...])
    h = _bn_relu(h, g1_ref[...], be1_ref[...])
    h2 = jnp.dot(h, w2_ref[...], preferred_element_type=jnp.float32) + b2_ref[...]
    h2 = _bn_relu(h2, g2_ref[...], be2_ref[...])
    out_ref[...] = (jnp.dot(h2, w3_ref[...], preferred_element_type=jnp.float32)
                    + b3_ref[...])


_mlp = pl.pallas_call(
    _mlp_body,
    out_shape=jax.ShapeDtypeStruct((BATCH, 1), jnp.float32),
)


def kernel(user_input, book_input, user_table, book_table,
           W1, b1, g1, be1, W2, b2, g2, be2, W3, b3):
    utab2 = user_table.reshape(-1, 2 * EMBED)
    btab2 = book_table.reshape(-1, 2 * EMBED)
    u_rows, v_rows = _make_gather()(
        user_input >> 1, book_input >> 1, utab2, btab2)
    out = _mlp(u_rows, v_rows,
               (user_input & 1).reshape(-1, 1), (book_input & 1).reshape(-1, 1),
               W1[:EMBED], W1[EMBED:],
               b1.reshape(1, -1), g1.reshape(1, -1), be1.reshape(1, -1),
               W2, b2.reshape(1, -1), g2.reshape(1, -1), be2.reshape(1, -1),
               W3, b3.reshape(1, 1))
    return out.reshape(BATCH)


# speedup vs baseline: 1.4645x; 1.4645x over previous
import functools

import jax
import jax.numpy as jnp
from jax import lax
from jax.experimental import pallas as pl
from jax.experimental.pallas import tpu as pltpu
from jax.experimental.pallas import tpu_sc as plsc

BATCH = 16384
EMBED = 64
NC = 2
NS = 16
NW = NC * NS
B_PER_W = BATCH // NW
PB = 32
NPH = B_PER_W // PB
L = 16


def _gather_body(idx_hbm, eye_hbm, tab, out, idx_vm, eye_vm, blocks, rows,
                 sem):
    wid = lax.axis_index("s") * NC + lax.axis_index("c")
    base = wid * B_PER_W
    pltpu.sync_copy(idx_hbm.at[pl.ds(base, B_PER_W)], idx_vm)
    pltpu.sync_copy(eye_hbm, eye_vm)

    def scalar_idx(k):
        start = pl.multiple_of((k // L) * L, L)
        chunk = idx_vm[pl.ds(start, L)]
        mask = eye_vm[k % L, pl.ds(0, L)]
        return jnp.max(chunk * mask)

    for p in range(NPH):
        def fire(j, _, p=p):
            i = scalar_idx(p * PB + j)
            blk = pl.multiple_of((i >> 3) * 8, 8)
            pltpu.async_copy(tab.at[pl.ds(blk, 8)], blocks.at[j], sem)
            return 0

        def drain_select(j, _, p=p):
            pltpu.make_async_copy(tab.at[pl.ds(0, 8)],
                                  blocks.at[j], sem).wait()
            k = p * PB + j
            i = scalar_idx(k)
            rm = i & 7
            for c in range(EMBED // L):
                rows[k, pl.ds(c * L, L)] = blocks[j, rm, pl.ds(c * L, L)]
            return 0

        lax.fori_loop(0, PB, fire, 0)
        lax.fori_loop(0, PB, drain_select, 0)
    pltpu.sync_copy(rows, out.at[pl.ds(base, B_PER_W)])


@functools.cache
def _make_gather():
    return functools.partial(
        pl.kernel,
        mesh=plsc.VectorSubcoreMesh(core_axis_name="c", subcore_axis_name="s"),
        compiler_params=pltpu.CompilerParams(needs_layout_passes=False),
        out_type=jax.ShapeDtypeStruct((BATCH, 2 * EMBED), jnp.float32),
        scratch_types=[
            pltpu.VMEM((B_PER_W,), jnp.int32),
            pltpu.VMEM((L, 2 * EMBED), jnp.int32),
            pltpu.VMEM((PB, 8, EMBED), jnp.float32),
            pltpu.VMEM((B_PER_W, 2 * EMBED), jnp.float32),
            pltpu.SemaphoreType.DMA,
        ],
    )(_gather_body)


def _bn_relu(h, g, be, eps=1e-5):
    mean = jnp.mean(h, axis=0, keepdims=True)
    c = h - mean
    var = jnp.mean(c * c, axis=0, keepdims=True)
    return jnp.maximum(c * lax.rsqrt(var + eps) * g + be, 0.0)


def _mlp_body(u_ref, v_ref, w1a_ref, w1b_ref, b1_ref, g1_ref, be1_ref,
              w2_ref, b2_ref, g2_ref, be2_ref, w3_ref, b3_ref, out_ref):
    h = (jnp.dot(u_ref[:, :EMBED], w1a_ref[...], preferred_element_type=jnp.float32)
         + jnp.dot(v_ref[:, :EMBED], w1b_ref[...], preferred_element_type=jnp.float32)
         + b1_ref[...])
    h = _bn_relu(h, g1_ref[...], be1_ref[...])
    h2 = jnp.dot(h, w2_ref[...], preferred_element_type=jnp.float32) + b2_ref[...]
    h2 = _bn_relu(h2, g2_ref[...], be2_ref[...])
    out_ref[...] = (jnp.dot(h2, w3_ref[...], preferred_element_type=jnp.float32)
                    + b3_ref[...])


_mlp = pl.pallas_call(
    _mlp_body,
    out_shape=jax.ShapeDtypeStruct((BATCH, 1), jnp.float32),
)


def kernel(user_input, book_input, user_table, book_table,
           W1, b1, g1, be1, W2, b2, g2, be2, W3, b3):
    gather = _make_gather()
    eye = jnp.eye(L, 2 * EMBED, dtype=jnp.int32)
    u_rows = gather(user_input, eye, user_table)
    v_rows = gather(book_input, eye, book_table)
    out = _mlp(u_rows, v_rows, W1[:EMBED], W1[EMBED:],
               b1.reshape(1, -1), g1.reshape(1, -1), be1.reshape(1, -1),
               W2, b2.reshape(1, -1), g2.reshape(1, -1), be2.reshape(1, -1),
               W3, b3.reshape(1, 1))
    return out.reshape(BATCH)


# speedup vs baseline: 1.4748x; 1.0070x over previous
import functools

import jax
import jax.numpy as jnp
from jax import lax
from jax.experimental import pallas as pl
from jax.experimental.pallas import tpu as pltpu
from jax.experimental.pallas import tpu_sc as plsc

BATCH = 16384
EMBED = 64
NC = 2
NS = 16
NW = NC * NS
B_PER_W = BATCH // NW
PB = 32
NPH = B_PER_W // PB
L = 16


def _gather_body(idx_hbm, eye_hbm, tab, out, idx_vm, eye_vm, blocks, rows,
                 sem):
    wid = lax.axis_index("s") * NC + lax.axis_index("c")
    base = wid * B_PER_W
    pltpu.sync_copy(idx_hbm.at[pl.ds(base, B_PER_W)], idx_vm)
    pltpu.sync_copy(eye_hbm, eye_vm)

    def scalar_idx(k):
        start = pl.multiple_of((k // L) * L, L)
        chunk = idx_vm[pl.ds(start, L)]
        mask = eye_vm[k % L, pl.ds(0, L)]
        return jnp.max(chunk * mask)

    for p in range(NPH):
        def fire(j, _, p=p):
            i = scalar_idx(p * PB + j)
            blk = pl.multiple_of((i >> 3) * 8, 8)
            pltpu.async_copy(tab.at[pl.ds(blk, 8)], blocks.at[j], sem)
            return 0

        def drain_select(j, _, p=p):
            pltpu.make_async_copy(tab.at[pl.ds(0, 8)],
                                  blocks.at[j], sem).wait()
            k = p * PB + j
            i = scalar_idx(k)
            rm = i & 7
            for c in range(EMBED // L):
                rows[k, pl.ds(c * L, L)] = blocks[j, rm, pl.ds(c * L, L)]
            return 0

        lax.fori_loop(0, PB, fire, 0)
        lax.fori_loop(0, PB, drain_select, 0)
    pltpu.sync_copy(rows, out.at[pl.ds(base, B_PER_W)])


@functools.cache
def _make_gather():
    return functools.partial(
        pl.kernel,
        mesh=plsc.VectorSubcoreMesh(core_axis_name="c", subcore_axis_name="s"),
        compiler_params=pltpu.CompilerParams(needs_layout_passes=False),
        out_type=jax.ShapeDtypeStruct((BATCH, 2 * EMBED), jnp.float32),
        scratch_types=[
            pltpu.VMEM((B_PER_W,), jnp.int32),
            pltpu.VMEM((L, 2 * EMBED), jnp.int32),
            pltpu.VMEM((PB, 8, EMBED), jnp.float32),
            pltpu.VMEM((B_PER_W, 2 * EMBED), jnp.float32),
            pltpu.SemaphoreType.DMA,
        ],
    )(_gather_body)


def _bn_relu(h, g, be, eps=1e-5):
    mean = jnp.mean(h, axis=0, keepdims=True)
    c = h - mean
    var = jnp.mean(c * c, axis=0, keepdims=True)
    return jnp.maximum(c * lax.rsqrt(var + eps) * g + be, 0.0)


def _mlp_body(u_ref, v_ref, w1a_ref, w1b_ref, b1_ref, g1_ref, be1_ref,
              w2_ref, b2_ref, g2_ref, be2_ref, w3_ref, b3_ref, out_ref):
    h = (jnp.dot(u_ref[:, :EMBED], w1a_ref[...], preferred_element_type=jnp.float32)
         + jnp.dot(v_ref[:, :EMBED], w1b_ref[...], preferred_element_type=jnp.float32)
         + b1_ref[...])
    h = _bn_relu(h, g1_ref[...], be1_ref[...])
    h2 = jnp.dot(h, w2_ref[...], preferred_element_type=jnp.float32) + b2_ref[...]
    h2 = _bn_relu(h2, g2_ref[...], be2_ref[...])
    out_ref[...] = (jnp.dot(h2, w3_ref[...], preferred_element_type=jnp.float32)
                    + b3_ref[...])


_mlp = pl.pallas_call(
    _mlp_body,
    out_shape=jax.ShapeDtypeStruct((BATCH, 1), jnp.float32),
)


def kernel(user_input, book_input, user_table, book_table,
           W1, b1, g1, be1, W2, b2, g2, be2, W3, b3):
    gather = _make_gather()
    eye = jnp.eye(L, 2 * EMBED, dtype=jnp.int32)
    v_rows = gather(book_input, eye, book_table)
    u_rows = gather(user_input, eye, user_table)
    out = _mlp(u_rows, v_rows, W1[:EMBED], W1[EMBED:],
               b1.reshape(1, -1), g1.reshape(1, -1), be1.reshape(1, -1),
               W2, b2.reshape(1, -1), g2.reshape(1, -1), be2.reshape(1, -1),
               W3, b3.reshape(1, 1))
    return out.reshape(BATCH)
